# baseline (speedup 1.0000x reference)
import functools

import jax
import jax.numpy as jnp
from jax import lax
from jax.experimental import pallas as pl
from jax.experimental.pallas import tpu as pltpu
from jax.experimental.pallas import tpu_sc as plsc

NC = 2
NS = 16
L = 16
NW = NC * NS

BATCH = 16384
D = 64
V = 1000000
B_PER_W = BATCH // NW
GROUPS = B_PER_W // L
CHUNK = 128
N_CHUNK = B_PER_W // CHUNK
ZROW = 16
BN = 32768

SLABS = 64
S_PER_TILE = SLABS * 128
S = NW * S_PER_TILE
TC_GRID = (V - S + BN - 1) // BN
TC_OFF = S // BN


def _tc_body(at_ref, bt_ref, wu_ref, wb_ref, zu_ref, zb_ref):
    zu_ref[...] = jnp.dot(wu_ref[...], at_ref[...],
                          preferred_element_type=jnp.float32)[0]
    zb_ref[...] = jnp.dot(wb_ref[...], bt_ref[...],
                          preferred_element_type=jnp.float32)[0]


@functools.cache
def _tc_matvec():
    return pl.pallas_call(
        _tc_body,
        grid=(TC_GRID,),
        in_specs=[
            pl.BlockSpec((D, BN), lambda i: (0, i + TC_OFF)),
            pl.BlockSpec((D, BN), lambda i: (0, i + TC_OFF)),
            pl.BlockSpec((1, D), lambda i: (0, 0)),
            pl.BlockSpec((1, D), lambda i: (0, 0)),
        ],
        out_specs=[
            pl.BlockSpec((BN,), lambda i: (i,)),
            pl.BlockSpec((BN,), lambda i: (i,)),
        ],
        out_shape=[
            jax.ShapeDtypeStruct((V - S,), jnp.float32),
            jax.ShapeDtypeStruct((V - S,), jnp.float32),
        ],
    )


def _sc_mv_table(p_hbm, w_v, out_v, slab, sem_a, sem_b, base):
    def fire(s, p):
        c0 = pl.multiple_of(base + s * 128, 128)
        return [pltpu.async_copy(p_hbm.at[pl.ds(8 * t, 8), pl.ds(c0, 128)],
                                 slab.at[p, t], sem_a if p == 0 else sem_b)
                for t in range(8)]

    def drain(p):
        pltpu.make_async_copy(p_hbm.at[pl.ds(0, 8), pl.ds(0, 128)],
                              slab.at[p, 0], sem_a if p == 0 else sem_b).wait()

    def compute(s, p):
        acc = [jnp.zeros((L,), jnp.float32) for _ in range(8)]
        for k in range(D):
            wk = w_v[pl.ds(k * L, L)]
            for l in range(8):
                acc[l] = acc[l] + slab[p, k // 8, k % 8, pl.ds(l * L, L)] * wk
        for l in range(8):
            out_v[pl.ds(s * 128 + l * L, L)] = acc[l]

    def drain_all(p):
        for _ in range(8):
            drain(p)

    fire(0, 0)

    def step(i, _):
        e = 2 * i
        fire(e + 1, 1)
        drain_all(0)
        compute(e, 0)

        @pl.when(e + 2 < SLABS)
        def _():
            fire(e + 2, 0)

        drain_all(1)
        compute(e + 1, 1)
        return _

    lax.fori_loop(0, SLABS // 2, step, None)


def _sc_mv_body(pu_hbm, pb_hbm, wu_hbm, wb_hbm,
                zu_hbm, zb_hbm,
                wu_v, wb_v, out_u, out_b, slab, sem_a, sem_b):
    c = lax.axis_index("c")
    s = lax.axis_index("s")
    wid = s * NC + c
    base = wid * S_PER_TILE

    pltpu.sync_copy(wu_hbm, wu_v)
    pltpu.sync_copy(wb_hbm, wb_v)

    _sc_mv_table(pu_hbm, wu_v, out_u, slab, sem_a, sem_b, base)
    _sc_mv_table(pb_hbm, wb_v, out_b, slab, sem_a, sem_b, base)

    pltpu.sync_copy(out_u, zu_hbm.at[pl.ds(base, S_PER_TILE)])
    pltpu.sync_copy(out_b, zb_hbm.at[pl.ds(base, S_PER_TILE)])


@functools.cache
def _sc_matvec():
    return pl.kernel(
        _sc_mv_body,
        out_type=[jax.ShapeDtypeStruct((S,), jnp.float32),
                  jax.ShapeDtypeStruct((S,), jnp.float32)],
        mesh=plsc.VectorSubcoreMesh(core_axis_name="c", subcore_axis_name="s",
                                    num_cores=NC, num_subcores=NS),
        scratch_types=[
            pltpu.VMEM((D * L,), jnp.float32),
            pltpu.VMEM((D * L,), jnp.float32),
            pltpu.VMEM((S_PER_TILE,), jnp.float32),
            pltpu.VMEM((S_PER_TILE,), jnp.float32),
            pltpu.VMEM((2, 8, 8, 128), jnp.float32),
            pltpu.SemaphoreType.DMA,
            pltpu.SemaphoreType.DMA,
        ],
        compiler_params=pltpu.CompilerParams(needs_layout_passes=False,
                                             use_tc_tiling_on_sc=True),
    )


def _sc_body(zu_hbm, zb_hbm, uid_hbm, bid_hbm, bias_hbm,
             out_hbm,
             idx_u, idx_b, row_u, row_b, lane_u, lane_b,
             rows_u, rows_b, bias_v, out_v, sem):
    c = lax.axis_index("c")
    s = lax.axis_index("s")
    wid = s * NC + c
    base = wid * B_PER_W

    pltpu.sync_copy(uid_hbm.at[pl.ds(base, B_PER_W)], idx_u)
    pltpu.sync_copy(bid_hbm.at[pl.ds(base, B_PER_W)], idx_b)
    pltpu.sync_copy(bias_hbm, bias_v)

    for k in range(GROUPS):
        j, off = k // 8, (k % 8) * L
        vu = idx_u[pl.ds(k * L, L)]
        vb = idx_b[pl.ds(k * L, L)]
        row_u[j, pl.ds(off, L)] = vu >> 4
        row_b[j, pl.ds(off, L)] = vb >> 4
        lane_u[pl.ds(k * L, L)] = vu & 15
        lane_b[pl.ds(k * L, L)] = vb & 15

    descs = []
    for j in range(N_CHUNK):
        descs.append(pltpu.async_copy(
            zu_hbm.at[row_u.at[j]], rows_u.at[pl.ds(j * CHUNK, CHUNK)], sem))
        descs.append(pltpu.async_copy(
            zb_hbm.at[row_b.at[j]], rows_b.at[pl.ds(j * CHUNK, CHUNK)], sem))
    for d in descs:
        d.wait()

    iota16 = lax.iota(jnp.int32, L)

    def group(g, _):
        rids = iota16 + g * L
        vu = plsc.load_gather(rows_u, [rids, lane_u[pl.ds(g * L, L)]])
        vb = plsc.load_gather(rows_b, [rids, lane_b[pl.ds(g * L, L)]])
        out_v[pl.ds(g * L, L)] = vu + vb + bias_v[...]
        return _

    lax.fori_loop(0, GROUPS, group, None)

    pltpu.sync_copy(out_v, out_hbm.at[pl.ds(base, B_PER_W)])


@functools.cache
def _sc_gather():
    return pl.kernel(
        _sc_body,
        out_type=jax.ShapeDtypeStruct((BATCH,), jnp.float32),
        mesh=plsc.VectorSubcoreMesh(core_axis_name="c", subcore_axis_name="s",
                                    num_cores=NC, num_subcores=NS),
        scratch_types=[
            pltpu.VMEM((B_PER_W,), jnp.int32),
            pltpu.VMEM((B_PER_W,), jnp.int32),
            pltpu.VMEM((N_CHUNK, CHUNK), jnp.int32),
            pltpu.VMEM((N_CHUNK, CHUNK), jnp.int32),
            pltpu.VMEM((B_PER_W,), jnp.int32),
            pltpu.VMEM((B_PER_W,), jnp.int32),
            pltpu.VMEM((B_PER_W, ZROW), jnp.float32),
            pltpu.VMEM((B_PER_W, ZROW), jnp.float32),
            pltpu.VMEM((L,), jnp.float32),
            pltpu.VMEM((B_PER_W,), jnp.float32),
            pltpu.SemaphoreType.DMA,
        ],
        compiler_params=pltpu.CompilerParams(needs_layout_passes=False,
                                             use_tc_tiling_on_sc=False),
    )


@jax.jit
def kernel(user_ids, book_ids, user_table, book_table, fc_w, fc_b):
    uid = user_ids.astype(jnp.int32)
    bid = book_ids.astype(jnp.int32)
    wu = fc_w[:D, 0].reshape(1, D).astype(jnp.float32)
    wb = fc_w[D:, 0].reshape(1, D).astype(jnp.float32)
    wu_flat = jnp.broadcast_to(fc_w[:D], (D, L)).reshape(D * L).astype(jnp.float32)
    wb_flat = jnp.broadcast_to(fc_w[D:], (D, L)).reshape(D * L).astype(jnp.float32)
    at = user_table.T
    bt = book_table.T
    zu_tc, zb_tc = _tc_matvec()(at, bt, wu, wb)
    zu_sc, zb_sc = _sc_matvec()(at, bt, wu_flat, wb_flat)
    zu2 = jnp.concatenate([zu_sc, zu_tc]).reshape(V // ZROW, ZROW)
    zb2 = jnp.concatenate([zb_sc, zb_tc]).reshape(V // ZROW, ZROW)
    bias = jnp.full((L,), fc_b[0], jnp.float32)
    return _sc_gather()(zu2, zb2, uid, bid, bias)


# speedup vs baseline: 1.4118x; 1.4118x over previous
import functools

import jax
import jax.numpy as jnp
from jax import lax
from jax.experimental import pallas as pl
from jax.experimental.pallas import tpu as pltpu
from jax.experimental.pallas import tpu_sc as plsc

NC = 2
NS = 16
L = 16
NW = NC * NS

BATCH = 16384
D = 64
V = 1000000
B_PER_W = BATCH // NW
GROUPS = B_PER_W // L
CHUNK = 128
N_CHUNK = B_PER_W // CHUNK
ZROW = 16
BN = 49152
GRID = (V + BN - 1) // BN


def _tc_body(at_ref, bt_ref, wu_ref, wb_ref, zu_ref, zb_ref):
    zu_ref[...] = jnp.dot(wu_ref[...], at_ref[...],
                          preferred_element_type=jnp.float32)[0]
    zb_ref[...] = jnp.dot(wb_ref[...], bt_ref[...],
                          preferred_element_type=jnp.float32)[0]


@functools.cache
def _tc_matvec():
    return pl.pallas_call(
        _tc_body,
        grid=(GRID,),
        in_specs=[
            pl.BlockSpec((D, BN), lambda i: (0, i)),
            pl.BlockSpec((D, BN), lambda i: (0, i)),
            pl.BlockSpec((1, D), lambda i: (0, 0)),
            pl.BlockSpec((1, D), lambda i: (0, 0)),
        ],
        out_specs=[
            pl.BlockSpec((BN,), lambda i: (i,)),
            pl.BlockSpec((BN,), lambda i: (i,)),
        ],
        out_shape=[
            jax.ShapeDtypeStruct((V,), jnp.float32),
            jax.ShapeDtypeStruct((V,), jnp.float32),
        ],
    )


def _sc_body(zu_hbm, zb_hbm, uid_hbm, bid_hbm, bias_hbm,
             out_hbm,
             idx_u, idx_b, row_u, row_b, lane_u, lane_b,
             rows_u, rows_b, bias_v, out_v, sem):
    c = lax.axis_index("c")
    s = lax.axis_index("s")
    wid = s * NC + c
    base = wid * B_PER_W

    pltpu.sync_copy(uid_hbm.at[pl.ds(base, B_PER_W)], idx_u)
    pltpu.sync_copy(bid_hbm.at[pl.ds(base, B_PER_W)], idx_b)
    pltpu.sync_copy(bias_hbm, bias_v)

    for k in range(GROUPS):
        j, off = k // 8, (k % 8) * L
        vu = idx_u[pl.ds(k * L, L)]
        vb = idx_b[pl.ds(k * L, L)]
        row_u[j, pl.ds(off, L)] = vu >> 4
        row_b[j, pl.ds(off, L)] = vb >> 4
        lane_u[pl.ds(k * L, L)] = vu & 15
        lane_b[pl.ds(k * L, L)] = vb & 15

    descs = []
    for j in range(N_CHUNK):
        descs.append(pltpu.async_copy(
            zu_hbm.at[row_u.at[j]], rows_u.at[pl.ds(j * CHUNK, CHUNK)], sem))
        descs.append(pltpu.async_copy(
            zb_hbm.at[row_b.at[j]], rows_b.at[pl.ds(j * CHUNK, CHUNK)], sem))
    for d in descs:
        d.wait()

    iota16 = lax.iota(jnp.int32, L)

    def group(g, _):
        rids = iota16 + g * L
        vu = plsc.load_gather(rows_u, [rids, lane_u[pl.ds(g * L, L)]])
        vb = plsc.load_gather(rows_b, [rids, lane_b[pl.ds(g * L, L)]])
        out_v[pl.ds(g * L, L)] = vu + vb + bias_v[...]
        return _

    lax.fori_loop(0, GROUPS, group, None)

    pltpu.sync_copy(out_v, out_hbm.at[pl.ds(base, B_PER_W)])


@functools.cache
def _sc_gather():
    return pl.kernel(
        _sc_body,
        out_type=jax.ShapeDtypeStruct((BATCH,), jnp.float32),
        mesh=plsc.VectorSubcoreMesh(core_axis_name="c", subcore_axis_name="s",
                                    num_cores=NC, num_subcores=NS),
        scratch_types=[
            pltpu.VMEM((B_PER_W,), jnp.int32),
            pltpu.VMEM((B_PER_W,), jnp.int32),
            pltpu.VMEM((N_CHUNK, CHUNK), jnp.int32),
            pltpu.VMEM((N_CHUNK, CHUNK), jnp.int32),
            pltpu.VMEM((B_PER_W,), jnp.int32),
            pltpu.VMEM((B_PER_W,), jnp.int32),
            pltpu.VMEM((B_PER_W, ZROW), jnp.float32),
            pltpu.VMEM((B_PER_W, ZROW), jnp.float32),
            pltpu.VMEM((L,), jnp.float32),
            pltpu.VMEM((B_PER_W,), jnp.float32),
            pltpu.SemaphoreType.DMA,
        ],
        compiler_params=pltpu.CompilerParams(needs_layout_passes=False,
                                             use_tc_tiling_on_sc=False),
    )


@jax.jit
def kernel(user_ids, book_ids, user_table, book_table, fc_w, fc_b):
    uid = user_ids.astype(jnp.int32)
    bid = book_ids.astype(jnp.int32)
    wu = fc_w[:D, 0].reshape(1, D).astype(jnp.float32)
    wb = fc_w[D:, 0].reshape(1, D).astype(jnp.float32)
    zu, zb = _tc_matvec()(user_table.T, book_table.T, wu, wb)
    zu2 = zu.reshape(V // ZROW, ZROW)
    zb2 = zb.reshape(V // ZROW, ZROW)
    bias = jnp.full((L,), fc_b[0], jnp.float32)
    return _sc_gather()(zu2, zb2, uid, bid, bias)


# speedup vs baseline: 1.4617x; 1.0354x over previous
import functools

import jax
import jax.numpy as jnp
from jax import lax
from jax.experimental import pallas as pl
from jax.experimental.pallas import tpu as pltpu
from jax.experimental.pallas import tpu_sc as plsc

NC = 2
NS = 16
L = 16
NW = NC * NS

BATCH = 16384
D = 64
V = 1000000
B_PER_W = BATCH // NW
GROUPS = B_PER_W // L
CHUNK = 128
N_CHUNK = B_PER_W // CHUNK
ZROW = 16
BN = 32768
GRID = (V + BN - 1) // BN


def _tc_body(at_ref, bt_ref, wu_ref, wb_ref, zu_ref, zb_ref):
    zu_ref[...] = jnp.dot(wu_ref[...], at_ref[...],
                          preferred_element_type=jnp.float32)[0]
    zb_ref[...] = jnp.dot(wb_ref[...], bt_ref[...],
                          preferred_element_type=jnp.float32)[0]


@functools.cache
def _tc_matvec():
    return pl.pallas_call(
        _tc_body,
        grid=(GRID,),
        in_specs=[
            pl.BlockSpec((D, BN), lambda i: (0, i)),
            pl.BlockSpec((D, BN), lambda i: (0, i)),
            pl.BlockSpec((1, D), lambda i: (0, 0)),
            pl.BlockSpec((1, D), lambda i: (0, 0)),
        ],
        out_specs=[
            pl.BlockSpec((BN,), lambda i: (i,)),
            pl.BlockSpec((BN,), lambda i: (i,)),
        ],
        out_shape=[
            jax.ShapeDtypeStruct((V,), jnp.float32),
            jax.ShapeDtypeStruct((V,), jnp.float32),
        ],
    )


def _sc_body(zu_hbm, zb_hbm, uid_hbm, bid_hbm, bias_hbm,
             out_hbm,
             idx_u, idx_b, row_u, row_b, lane_u, lane_b,
             rows_u, rows_b, bias_v, out_v, sem):
    c = lax.axis_index("c")
    s = lax.axis_index("s")
    wid = s * NC + c
    base = wid * B_PER_W

    pltpu.sync_copy(uid_hbm.at[pl.ds(base, B_PER_W)], idx_u)
    pltpu.sync_copy(bid_hbm.at[pl.ds(base, B_PER_W)], idx_b)
    pltpu.sync_copy(bias_hbm, bias_v)

    for k in range(GROUPS):
        j, off = k // 8, (k % 8) * L
        vu = idx_u[pl.ds(k * L, L)]
        vb = idx_b[pl.ds(k * L, L)]
        row_u[j, pl.ds(off, L)] = vu >> 4
        row_b[j, pl.ds(off, L)] = vb >> 4
        lane_u[pl.ds(k * L, L)] = vu & 15
        lane_b[pl.ds(k * L, L)] = vb & 15

    descs = []
    for j in range(N_CHUNK):
        descs.append(pltpu.async_copy(
            zu_hbm.at[row_u.at[j]], rows_u.at[pl.ds(j * CHUNK, CHUNK)], sem))
        descs.append(pltpu.async_copy(
            zb_hbm.at[row_b.at[j]], rows_b.at[pl.ds(j * CHUNK, CHUNK)], sem))
    for d in descs:
        d.wait()

    iota16 = lax.iota(jnp.int32, L)

    def group(g, _):
        rids = iota16 + g * L
        vu = plsc.load_gather(rows_u, [rids, lane_u[pl.ds(g * L, L)]])
        vb = plsc.load_gather(rows_b, [rids, lane_b[pl.ds(g * L, L)]])
        out_v[pl.ds(g * L, L)] = vu + vb + bias_v[...]
        return _

    lax.fori_loop(0, GROUPS, group, None)

    pltpu.sync_copy(out_v, out_hbm.at[pl.ds(base, B_PER_W)])


@functools.cache
def _sc_gather():
    return pl.kernel(
        _sc_body,
        out_type=jax.ShapeDtypeStruct((BATCH,), jnp.float32),
        mesh=plsc.VectorSubcoreMesh(core_axis_name="c", subcore_axis_name="s",
                                    num_cores=NC, num_subcores=NS),
        scratch_types=[
            pltpu.VMEM((B_PER_W,), jnp.int32),
            pltpu.VMEM((B_PER_W,), jnp.int32),
            pltpu.VMEM((N_CHUNK, CHUNK), jnp.int32),
            pltpu.VMEM((N_CHUNK, CHUNK), jnp.int32),
            pltpu.VMEM((B_PER_W,), jnp.int32),
            pltpu.VMEM((B_PER_W,), jnp.int32),
            pltpu.VMEM((B_PER_W, ZROW), jnp.float32),
            pltpu.VMEM((B_PER_W, ZROW), jnp.float32),
            pltpu.VMEM((L,), jnp.float32),
            pltpu.VMEM((B_PER_W,), jnp.float32),
            pltpu.SemaphoreType.DMA,
        ],
        compiler_params=pltpu.CompilerParams(needs_layout_passes=False,
                                             use_tc_tiling_on_sc=False),
    )


@jax.jit
def kernel(user_ids, book_ids, user_table, book_table, fc_w, fc_b):
    uid = user_ids.astype(jnp.int32)
    bid = book_ids.astype(jnp.int32)
    wu = fc_w[:D, 0].reshape(1, D).astype(jnp.float32)
    wb = fc_w[D:, 0].reshape(1, D).astype(jnp.float32)
    zu, zb = _tc_matvec()(user_table.T, book_table.T, wu, wb)
    zu2 = zu.reshape(V // ZROW, ZROW)
    zb2 = zb.reshape(V // ZROW, ZROW)
    bias = jnp.full((L,), fc_b[0], jnp.float32)
    return _sc_gather()(zu2, zb2, uid, bid, bias)
